# baseline (speedup 1.0000x reference)
import functools

import jax
import jax.numpy as jnp
from jax import lax
from jax.experimental import pallas as pl
from jax.experimental.pallas import tpu as pltpu
from jax.experimental.pallas import tpu_sc as plsc

N = 10000
E = 320000
D = 128
DEPTH = 3
BATCH = 10
GPN = N // BATCH

NC = 2
NS = 16
NW = NC * NS
CH = 128
CPW = 80
EPC = NW * CPW * CH
NP = 10240
RPT = NP // NS
ZR = 128
NRING = 4


def _seg_partials(h, src2d, dst2d):
  mesh = plsc.VectorSubcoreMesh(core_axis_name="c", subcore_axis_name="s")

  @functools.partial(
      pl.kernel,
      mesh=mesh,
      out_type=jax.ShapeDtypeStruct((NC * NP, D), jnp.float32),
      scratch_types=[
          pltpu.VMEM((2, CH), jnp.int32),
          pltpu.VMEM((2, CH), jnp.int32),
          pltpu.VMEM((CH, D), jnp.float32),
          pltpu.VMEM((CH, D), jnp.float32),
          pltpu.VMEM_SHARED((NP, D), jnp.float32),
          pltpu.SemaphoreType.DMA,
          pltpu.SemaphoreType.DMA,
          pltpu.SemaphoreType.DMA,
      ],
  )
  def k(h_hbm, src_hbm, dst_hbm, out_hbm,
        sidx, didx, rows0, rows1, acc, sem_i, sem0, sem1):
    cid = lax.axis_index("c")
    sid = lax.axis_index("s")
    wid = sid * NC + cid
    ebase = wid * CPW * CH

    def iload(j, s):
      pltpu.sync_copy(src_hbm.at[pl.ds(ebase + j * CH, CH)], sidx.at[s])
      pltpu.sync_copy(dst_hbm.at[pl.ds(ebase + j * CH, CH)], didx.at[s])


    zeros16 = jnp.zeros((16,), jnp.float32)

    def zb(i, carry):
      r = i // (D // 16)
      c = i % (D // 16)
      rows0[r, pl.ds(c * 16, 16)] = zeros16
      return carry

    lax.fori_loop(0, CH * (D // 16), zb, 0)

    def zc(i, carry):
      pltpu.sync_copy(rows0, acc.at[pl.ds(sid * RPT + i * ZR, ZR)])
      return carry

    lax.fori_loop(0, RPT // ZR, zc, 0)
    plsc.subcore_barrier()

    def gstart(s, buf, sem):
      pltpu.make_async_copy(h_hbm.at[sidx.at[s]], buf, sem).start()

    def gwait(s, buf, sem):
      pltpu.make_async_copy(h_hbm.at[sidx.at[s]], buf, sem).wait()

    iload(0, 0)
    gstart(0, rows0, sem0)
    last = CPW // 2 - 1

    def body(jj, carry):
      j = jj * 2
      iload(j + 1, 1)
      gstart(1, rows1, sem1)
      gwait(0, rows0, sem0)
      pltpu.sync_copy(rows0, acc.at[didx.at[0]], add=True)

      @pl.when(jj < last)
      def _():
        iload(j + 2, 0)
        gstart(0, rows0, sem0)

      gwait(1, rows1, sem1)
      pltpu.sync_copy(rows1, acc.at[didx.at[1]], add=True)
      return carry

    lax.fori_loop(0, CPW // 2, body, 0)
    plsc.subcore_barrier()

    def wo(i, carry):
      r0 = sid * RPT + i * ZR
      pltpu.sync_copy(acc.at[pl.ds(r0, ZR)], rows0)
      pltpu.sync_copy(rows0, out_hbm.at[pl.ds(cid * NP + r0, ZR)])
      return carry

    lax.fori_loop(0, RPT // ZR, wo, 0)

  return k(h, src2d, dst2d)


def _layer_body(h_ref, parts_ref, w1_ref, b1_ref, g1_ref, bb1_ref,
                w2_ref, b2_ref, g2_ref, bb2_ref, o_ref):
  z = h_ref[...] + parts_ref[0] + parts_ref[1]
  z = jnp.dot(z, w1_ref[...], preferred_element_type=jnp.float32) + b1_ref[...]
  mu = jnp.mean(z, axis=0, keepdims=True)
  var = jnp.mean(jnp.square(z - mu), axis=0, keepdims=True)
  z = (z - mu) * lax.rsqrt(var + 1e-5) * g1_ref[...] + bb1_ref[...]
  z = jnp.maximum(z, 0.0)
  z = jnp.dot(z, w2_ref[...], preferred_element_type=jnp.float32) + b2_ref[...]
  mu = jnp.mean(z, axis=0, keepdims=True)
  var = jnp.mean(jnp.square(z - mu), axis=0, keepdims=True)
  z = (z - mu) * lax.rsqrt(var + 1e-5) * g2_ref[...] + bb2_ref[...]
  o_ref[...] = jnp.maximum(z, 0.0)


def _layer_tc(h, parts, w1, b1, g1, bb1, w2, b2, g2, bb2):
  r = lambda a: a.reshape(1, D)
  wsp = lambda s: pl.BlockSpec(s, lambda i: (0,) * len(s))
  return pl.pallas_call(
      _layer_body,
      grid=(1,),
      in_specs=[
          wsp((N, D)),
          wsp((2, N, D)),
          wsp((D, D)), wsp((1, D)), wsp((1, D)), wsp((1, D)),
          wsp((D, D)), wsp((1, D)), wsp((1, D)), wsp((1, D)),
      ],
      out_specs=wsp((N, D)),
      out_shape=jax.ShapeDtypeStruct((N, D), jnp.float32),
  )(h, parts.reshape(2, NP, D), w1, r(b1), r(g1), r(bb1),
    w2, r(b2), r(g2), r(bb2))


def _head_body(x_ref, x1_ref, x2_ref, x3_ref, jkw_ref, jkb_ref,
               linw_ref, linb_ref, vw_ref, vb_ref, piw_ref, pib_ref,
               pi_ref, v_ref):
  xb = x_ref[0]
  hcat = jnp.concatenate([x1_ref[0], x2_ref[0], x3_ref[0]], axis=1)
  hcat = jnp.dot(hcat, jkw_ref[...],
                 preferred_element_type=jnp.float32) + jkb_ref[...]
  xfull = jnp.concatenate([xb, hcat], axis=1)
  feat = jnp.dot(xfull, linw_ref[...],
                 preferred_element_type=jnp.float32) + linb_ref[...]
  piv = jnp.sum(feat * piw_ref[...], axis=1) + pib_ref[0, 0]
  m = jnp.max(piv)
  lse = jnp.log(jnp.sum(jnp.exp(piv - m))) + m
  pi_ref[0, 0, :] = piv - lse
  fm = jnp.mean(feat, axis=0, keepdims=True)
  v = jnp.dot(fm, vw_ref[...], preferred_element_type=jnp.float32) + vb_ref[...]
  vm = jnp.max(v)
  vlse = jnp.log(jnp.sum(jnp.exp(v - vm))) + vm
  v_ref[0] = v - vlse


def _head_tc(x, x1, x2, x3, jk_w, jk_b, lin_w, lin_b, v_w, v_b, pi_w, pi_b):
  g3 = lambda g: (g, 0, 0)
  w0 = lambda g: (0, 0)
  DD = DEPTH * D
  return pl.pallas_call(
      _head_body,
      grid=(BATCH,),
      in_specs=[
          pl.BlockSpec((1, GPN, D), g3),
          pl.BlockSpec((1, GPN, D), g3),
          pl.BlockSpec((1, GPN, D), g3),
          pl.BlockSpec((1, GPN, D), g3),
          pl.BlockSpec((DD, DD), w0),
          pl.BlockSpec((1, DD), w0),
          pl.BlockSpec((D + DD, 32), w0),
          pl.BlockSpec((1, 32), w0),
          pl.BlockSpec((32, 3), w0),
          pl.BlockSpec((1, 3), w0),
          pl.BlockSpec((1, 32), w0),
          pl.BlockSpec((1, 1), w0),
      ],
      out_specs=[
          pl.BlockSpec((1, 1, GPN), g3),
          pl.BlockSpec((1, 1, 3), g3),
      ],
      out_shape=[
          jax.ShapeDtypeStruct((BATCH, 1, GPN), jnp.float32),
          jax.ShapeDtypeStruct((BATCH, 1, 3), jnp.float32),
      ],
  )(x.reshape(BATCH, GPN, D), x1.reshape(BATCH, GPN, D),
    x2.reshape(BATCH, GPN, D), x3.reshape(BATCH, GPN, D),
    jk_w, jk_b.reshape(1, DD), lin_w, lin_b.reshape(1, 32),
    v_w, v_b.reshape(1, 3), pi_w.reshape(1, 32), pi_b.reshape(1, 1))


def kernel(x, edge_index, batch_size, gin_W1, gin_b1, gin_bn_g, gin_bn_b,
           gin_W2, gin_b2, norm_g, norm_b, jk_W, jk_b, lin_W, lin_b,
           v_W, v_b, pi_W, pi_b):
  pad = EPC - E
  src2d = jnp.concatenate([edge_index[0], jnp.zeros((pad,), jnp.int32)])
  dst2d = jnp.concatenate([edge_index[1], jnp.full((pad,), NP - 1, jnp.int32)])
  h = x
  xs = []
  for i in range(DEPTH):
    parts = _seg_partials(h, src2d, dst2d)
    h = _layer_tc(h, parts, gin_W1[i], gin_b1[i],
                  gin_bn_g[i], gin_bn_b[i], gin_W2[i], gin_b2[i],
                  norm_g[i], norm_b[i])
    xs.append(h)
  pi, v = _head_tc(x, xs[0], xs[1], xs[2], jk_W, jk_b, lin_W, lin_b,
                   v_W, v_b, pi_W, pi_b)
  return (pi.reshape(BATCH, GPN), v.reshape(BATCH, 3))


# speedup vs baseline: 2.7743x; 2.7743x over previous
import functools

import jax
import jax.numpy as jnp
from jax import lax
from jax.experimental import pallas as pl
from jax.experimental.pallas import tpu as pltpu
from jax.experimental.pallas import tpu_sc as plsc

N = 10000
E = 320000
D = 128
DEPTH = 3
BATCH = 10
GPN = N // BATCH

NC = 2
NS = 16
NW = NC * NS
CH = 128
CPW = 80
EPC = NW * CPW * CH
NP = 10240
RPT = NP // NS
ZR = 128
NRING = 4


def _seg_partials(h, src2d, dst2d):
  mesh = plsc.VectorSubcoreMesh(core_axis_name="c", subcore_axis_name="s")

  @functools.partial(
      pl.kernel,
      mesh=mesh,
      out_type=jax.ShapeDtypeStruct((NC * NP, D), jnp.float32),
      scratch_types=[
          pltpu.VMEM((2, CH), jnp.int32),
          pltpu.VMEM((2, CH), jnp.int32),
          pltpu.VMEM((CH, D), jnp.float32),
          pltpu.VMEM((CH, D), jnp.float32),
          pltpu.VMEM_SHARED((NP, D), jnp.float32),
          pltpu.SemaphoreType.DMA,
          pltpu.SemaphoreType.DMA,
          pltpu.SemaphoreType.DMA,
      ],
  )
  def k(h_hbm, src_hbm, dst_hbm, out_hbm,
        sidx, didx, rows0, rows1, acc, sem_i, sem0, sem1):
    cid = lax.axis_index("c")
    sid = lax.axis_index("s")
    wid = sid * NC + cid
    ebase = wid * CPW * CH

    def iload(j, s):
      pltpu.sync_copy(src_hbm.at[pl.ds(ebase + j * CH, CH)], sidx.at[s])
      pltpu.sync_copy(dst_hbm.at[pl.ds(ebase + j * CH, CH)], didx.at[s])


    zeros16 = jnp.zeros((16,), jnp.float32)

    def zb(i, carry):
      r = i // (D // 16)
      c = i % (D // 16)
      rows0[r, pl.ds(c * 16, 16)] = zeros16
      return carry

    lax.fori_loop(0, CH * (D // 16), zb, 0)

    def zc(i, carry):
      pltpu.sync_copy(rows0, acc.at[pl.ds(sid * RPT + i * ZR, ZR)])
      return carry

    lax.fori_loop(0, RPT // ZR, zc, 0)
    plsc.subcore_barrier()

    def gstart(s, buf, sem):
      pltpu.make_async_copy(h_hbm.at[sidx.at[s]], buf, sem).start()

    def gwait(s, buf, sem):
      pltpu.make_async_copy(h_hbm.at[sidx.at[s]], buf, sem).wait()

    iload(0, 0)
    gstart(0, rows0, sem0)
    last = CPW // 2 - 1

    def body(jj, carry):
      j = jj * 2
      iload(j + 1, 1)
      gstart(1, rows1, sem1)
      gwait(0, rows0, sem0)
      pltpu.sync_copy(rows0, acc.at[didx.at[0]], add=True)

      @pl.when(jj < last)
      def _():
        iload(j + 2, 0)
        gstart(0, rows0, sem0)

      gwait(1, rows1, sem1)
      pltpu.sync_copy(rows1, acc.at[didx.at[1]], add=True)
      return carry

    lax.fori_loop(0, CPW // 2, body, 0)
    plsc.subcore_barrier()

    def wo(i, carry):
      r0 = sid * RPT + i * ZR
      pltpu.sync_copy(acc.at[pl.ds(r0, ZR)], rows0)
      pltpu.sync_copy(rows0, out_hbm.at[pl.ds(cid * NP + r0, ZR)])
      return carry

    lax.fori_loop(0, RPT // ZR, wo, 0)

  return k(h, src2d, dst2d)


def _layer_body(h_ref, parts_ref, w1_ref, b1_ref, g1_ref, bb1_ref,
                w2_ref, b2_ref, g2_ref, bb2_ref, o_ref):
  z = h_ref[...] + parts_ref[0] + parts_ref[1]
  z = jnp.dot(z, w1_ref[...], preferred_element_type=jnp.float32) + b1_ref[...]
  mu = jnp.mean(z, axis=0, keepdims=True)
  var = jnp.mean(jnp.square(z - mu), axis=0, keepdims=True)
  z = (z - mu) * lax.rsqrt(var + 1e-5) * g1_ref[...] + bb1_ref[...]
  z = jnp.maximum(z, 0.0)
  z = jnp.dot(z, w2_ref[...], preferred_element_type=jnp.float32) + b2_ref[...]
  mu = jnp.mean(z, axis=0, keepdims=True)
  var = jnp.mean(jnp.square(z - mu), axis=0, keepdims=True)
  z = (z - mu) * lax.rsqrt(var + 1e-5) * g2_ref[...] + bb2_ref[...]
  o_ref[...] = jnp.maximum(z, 0.0)


def _layer_tc(h, parts, w1, b1, g1, bb1, w2, b2, g2, bb2):
  r = lambda a: a.reshape(1, D)
  wsp = lambda s: pl.BlockSpec(s, lambda i: (0,) * len(s))
  return pl.pallas_call(
      _layer_body,
      grid=(1,),
      in_specs=[
          wsp((N, D)),
          wsp((2, N, D)),
          wsp((D, D)), wsp((1, D)), wsp((1, D)), wsp((1, D)),
          wsp((D, D)), wsp((1, D)), wsp((1, D)), wsp((1, D)),
      ],
      out_specs=wsp((N, D)),
      out_shape=jax.ShapeDtypeStruct((N, D), jnp.float32),
  )(h, parts.reshape(2, NP, D), w1, r(b1), r(g1), r(bb1),
    w2, r(b2), r(g2), r(bb2))


def _head_body(x_ref, x1_ref, x2_ref, x3_ref, jkw_ref, jkb_ref,
               linw_ref, linb_ref, vw_ref, vb_ref, piw_ref, pib_ref,
               pi_ref, v_ref):
  xb = x_ref[0]
  hcat = jnp.concatenate([x1_ref[0], x2_ref[0], x3_ref[0]], axis=1)
  hcat = jnp.dot(hcat, jkw_ref[...],
                 preferred_element_type=jnp.float32) + jkb_ref[...]
  xfull = jnp.concatenate([xb, hcat], axis=1)
  feat = jnp.dot(xfull, linw_ref[...],
                 preferred_element_type=jnp.float32) + linb_ref[...]
  piv = jnp.sum(feat * piw_ref[...], axis=1) + pib_ref[0, 0]
  m = jnp.max(piv)
  lse = jnp.log(jnp.sum(jnp.exp(piv - m))) + m
  pi_ref[0, 0, :] = piv - lse
  fm = jnp.mean(feat, axis=0, keepdims=True)
  v = jnp.dot(fm, vw_ref[...], preferred_element_type=jnp.float32) + vb_ref[...]
  vm = jnp.max(v)
  vlse = jnp.log(jnp.sum(jnp.exp(v - vm))) + vm
  v_ref[0] = v - vlse


def _head_tc(x, x1, x2, x3, jk_w, jk_b, lin_w, lin_b, v_w, v_b, pi_w, pi_b):
  g3 = lambda g: (g, 0, 0)
  w0 = lambda g: (0, 0)
  DD = DEPTH * D
  return pl.pallas_call(
      _head_body,
      grid=(BATCH,),
      in_specs=[
          pl.BlockSpec((1, GPN, D), g3),
          pl.BlockSpec((1, GPN, D), g3),
          pl.BlockSpec((1, GPN, D), g3),
          pl.BlockSpec((1, GPN, D), g3),
          pl.BlockSpec((DD, DD), w0),
          pl.BlockSpec((1, DD), w0),
          pl.BlockSpec((D + DD, 32), w0),
          pl.BlockSpec((1, 32), w0),
          pl.BlockSpec((32, 3), w0),
          pl.BlockSpec((1, 3), w0),
          pl.BlockSpec((1, 32), w0),
          pl.BlockSpec((1, 1), w0),
      ],
      out_specs=[
          pl.BlockSpec((1, 1, GPN), g3),
          pl.BlockSpec((1, 1, 3), g3),
      ],
      out_shape=[
          jax.ShapeDtypeStruct((BATCH, 1, GPN), jnp.float32),
          jax.ShapeDtypeStruct((BATCH, 1, 3), jnp.float32),
      ],
  )(x.reshape(BATCH, GPN, D), x1.reshape(BATCH, GPN, D),
    x2.reshape(BATCH, GPN, D), x3.reshape(BATCH, GPN, D),
    jk_w, jk_b.reshape(1, DD), lin_w, lin_b.reshape(1, 32),
    v_w, v_b.reshape(1, 3), pi_w.reshape(1, 32), pi_b.reshape(1, 1))


def kernel(x, edge_index, batch_size, gin_W1, gin_b1, gin_bn_g, gin_bn_b,
           gin_W2, gin_b2, norm_g, norm_b, jk_W, jk_b, lin_W, lin_b,
           v_W, v_b, pi_W, pi_b):
  pad = EPC - E
  ar = jnp.arange(pad, dtype=jnp.int32)
  src2d = jnp.concatenate([edge_index[0], ar % N])
  dst2d = jnp.concatenate([edge_index[1], N + (ar % (NP - N))])
  h = x
  xs = []
  for i in range(DEPTH):
    parts = _seg_partials(h, src2d, dst2d)
    h = _layer_tc(h, parts, gin_W1[i], gin_b1[i],
                  gin_bn_g[i], gin_bn_b[i], gin_W2[i], gin_b2[i],
                  norm_g[i], norm_b[i])
    xs.append(h)
  pi, v = _head_tc(x, xs[0], xs[1], xs[2], jk_W, jk_b, lin_W, lin_b,
                   v_W, v_b, pi_W, pi_b)
  return (pi.reshape(BATCH, GPN), v.reshape(BATCH, 3))


# speedup vs baseline: 3.6054x; 1.2996x over previous
import functools

import jax
import jax.numpy as jnp
from jax import lax
from jax.experimental import pallas as pl
from jax.experimental.pallas import tpu as pltpu
from jax.experimental.pallas import tpu_sc as plsc

N = 10000
E = 320000
D = 128
DEPTH = 3
BATCH = 10
GPN = N // BATCH

NC = 2
NS = 16
NW = NC * NS
CH = 128
CPW = 80
EPC = NW * CPW * CH
NP = 10240
RPT = NP // NS
ZR = 128
NRING = 4


def _seg_partials(h, src2d, dst2d):
  mesh = plsc.VectorSubcoreMesh(core_axis_name="c", subcore_axis_name="s")

  @functools.partial(
      pl.kernel,
      mesh=mesh,
      out_type=jax.ShapeDtypeStruct((NC * NP, D), jnp.float32),
      scratch_types=[
          pltpu.VMEM((NRING, CH), jnp.int32),
          pltpu.VMEM((NRING, CH), jnp.int32),
          pltpu.VMEM((CH, D), jnp.float32),
          pltpu.VMEM((CH, D), jnp.float32),
          pltpu.VMEM_SHARED((NP, D), jnp.float32),
          pltpu.SemaphoreType.DMA,
          pltpu.SemaphoreType.DMA,
          pltpu.SemaphoreType.DMA,
      ],
  )
  def k(h_hbm, src_hbm, dst_hbm, out_hbm,
        sidx, didx, rows0, rows1, acc, sem_i, sem0, sem1):
    cid = lax.axis_index("c")
    sid = lax.axis_index("s")
    wid = sid * NC + cid
    ebase = wid * CPW * CH

    def ifire(j):
      s = j % NRING
      pltpu.make_async_copy(
          src_hbm.at[pl.ds(ebase + j * CH, CH)], sidx.at[s], sem_i).start()
      pltpu.make_async_copy(
          dst_hbm.at[pl.ds(ebase + j * CH, CH)], didx.at[s], sem_i).start()

    def idrain(j):
      s = j % NRING
      pltpu.make_async_copy(
          src_hbm.at[pl.ds(ebase + j * CH, CH)], sidx.at[s], sem_i).wait()
      pltpu.make_async_copy(
          dst_hbm.at[pl.ds(ebase + j * CH, CH)], didx.at[s], sem_i).wait()

    for j in range(2):
      ifire(j)

    zeros16 = jnp.zeros((16,), jnp.float32)

    def zb(i, carry):
      r = i // (D // 16)
      c = i % (D // 16)
      rows0[r, pl.ds(c * 16, 16)] = zeros16
      return carry

    lax.fori_loop(0, CH * (D // 16), zb, 0)

    def zc(i, carry):
      pltpu.sync_copy(rows0, acc.at[pl.ds(sid * RPT + i * ZR, ZR)])
      return carry

    lax.fori_loop(0, RPT // ZR, zc, 0)
    for j in range(2):
      idrain(j)
    plsc.subcore_barrier()

    def gstart(j, buf, sem):
      pltpu.make_async_copy(h_hbm.at[sidx.at[j % NRING]], buf, sem).start()

    def gwait(j, buf, sem):
      pltpu.make_async_copy(h_hbm.at[sidx.at[j % NRING]], buf, sem).wait()

    gstart(0, rows0, sem0)
    last = CPW // 2 - 1

    def body(jj, carry):
      j = jj * 2

      @pl.when(jj < last)
      def _():
        ifire(j + 2)
        ifire(j + 3)

      gstart(j + 1, rows1, sem1)
      gwait(j, rows0, sem0)
      pltpu.sync_copy(rows0, acc.at[didx.at[j % NRING]], add=True)

      @pl.when(jj < last)
      def _():
        idrain(j + 2)
        idrain(j + 3)
        gstart(j + 2, rows0, sem0)

      gwait(j + 1, rows1, sem1)
      pltpu.sync_copy(rows1, acc.at[didx.at[(j + 1) % NRING]], add=True)
      return carry

    lax.fori_loop(0, CPW // 2, body, 0)
    plsc.subcore_barrier()

    def wo(i, carry):
      r0 = sid * RPT + i * ZR
      pltpu.sync_copy(acc.at[pl.ds(r0, ZR)], rows0)
      pltpu.sync_copy(rows0, out_hbm.at[pl.ds(cid * NP + r0, ZR)])
      return carry

    lax.fori_loop(0, RPT // ZR, wo, 0)

  return k(h, src2d, dst2d)


def _layer_body(h_ref, parts_ref, w1_ref, b1_ref, g1_ref, bb1_ref,
                w2_ref, b2_ref, g2_ref, bb2_ref, o_ref):
  z = h_ref[...] + parts_ref[0] + parts_ref[1]
  z = jnp.dot(z, w1_ref[...], preferred_element_type=jnp.float32) + b1_ref[...]
  mu = jnp.mean(z, axis=0, keepdims=True)
  var = jnp.mean(jnp.square(z - mu), axis=0, keepdims=True)
  z = (z - mu) * lax.rsqrt(var + 1e-5) * g1_ref[...] + bb1_ref[...]
  z = jnp.maximum(z, 0.0)
  z = jnp.dot(z, w2_ref[...], preferred_element_type=jnp.float32) + b2_ref[...]
  mu = jnp.mean(z, axis=0, keepdims=True)
  var = jnp.mean(jnp.square(z - mu), axis=0, keepdims=True)
  z = (z - mu) * lax.rsqrt(var + 1e-5) * g2_ref[...] + bb2_ref[...]
  o_ref[...] = jnp.maximum(z, 0.0)


def _layer_tc(h, parts, w1, b1, g1, bb1, w2, b2, g2, bb2):
  r = lambda a: a.reshape(1, D)
  wsp = lambda s: pl.BlockSpec(s, lambda i: (0,) * len(s))
  return pl.pallas_call(
      _layer_body,
      grid=(1,),
      in_specs=[
          wsp((N, D)),
          wsp((2, N, D)),
          wsp((D, D)), wsp((1, D)), wsp((1, D)), wsp((1, D)),
          wsp((D, D)), wsp((1, D)), wsp((1, D)), wsp((1, D)),
      ],
      out_specs=wsp((N, D)),
      out_shape=jax.ShapeDtypeStruct((N, D), jnp.float32),
  )(h, parts.reshape(2, NP, D), w1, r(b1), r(g1), r(bb1),
    w2, r(b2), r(g2), r(bb2))


def _head_body(x_ref, x1_ref, x2_ref, x3_ref, jkw_ref, jkb_ref,
               linw_ref, linb_ref, vw_ref, vb_ref, piw_ref, pib_ref,
               pi_ref, v_ref):
  xb = x_ref[0]
  hcat = jnp.concatenate([x1_ref[0], x2_ref[0], x3_ref[0]], axis=1)
  hcat = jnp.dot(hcat, jkw_ref[...],
                 preferred_element_type=jnp.float32) + jkb_ref[...]
  xfull = jnp.concatenate([xb, hcat], axis=1)
  feat = jnp.dot(xfull, linw_ref[...],
                 preferred_element_type=jnp.float32) + linb_ref[...]
  piv = jnp.sum(feat * piw_ref[...], axis=1) + pib_ref[0, 0]
  m = jnp.max(piv)
  lse = jnp.log(jnp.sum(jnp.exp(piv - m))) + m
  pi_ref[0, 0, :] = piv - lse
  fm = jnp.mean(feat, axis=0, keepdims=True)
  v = jnp.dot(fm, vw_ref[...], preferred_element_type=jnp.float32) + vb_ref[...]
  vm = jnp.max(v)
  vlse = jnp.log(jnp.sum(jnp.exp(v - vm))) + vm
  v_ref[0] = v - vlse


def _head_tc(x, x1, x2, x3, jk_w, jk_b, lin_w, lin_b, v_w, v_b, pi_w, pi_b):
  g3 = lambda g: (g, 0, 0)
  w0 = lambda g: (0, 0)
  DD = DEPTH * D
  return pl.pallas_call(
      _head_body,
      grid=(BATCH,),
      in_specs=[
          pl.BlockSpec((1, GPN, D), g3),
          pl.BlockSpec((1, GPN, D), g3),
          pl.BlockSpec((1, GPN, D), g3),
          pl.BlockSpec((1, GPN, D), g3),
          pl.BlockSpec((DD, DD), w0),
          pl.BlockSpec((1, DD), w0),
          pl.BlockSpec((D + DD, 32), w0),
          pl.BlockSpec((1, 32), w0),
          pl.BlockSpec((32, 3), w0),
          pl.BlockSpec((1, 3), w0),
          pl.BlockSpec((1, 32), w0),
          pl.BlockSpec((1, 1), w0),
      ],
      out_specs=[
          pl.BlockSpec((1, 1, GPN), g3),
          pl.BlockSpec((1, 1, 3), g3),
      ],
      out_shape=[
          jax.ShapeDtypeStruct((BATCH, 1, GPN), jnp.float32),
          jax.ShapeDtypeStruct((BATCH, 1, 3), jnp.float32),
      ],
  )(x.reshape(BATCH, GPN, D), x1.reshape(BATCH, GPN, D),
    x2.reshape(BATCH, GPN, D), x3.reshape(BATCH, GPN, D),
    jk_w, jk_b.reshape(1, DD), lin_w, lin_b.reshape(1, 32),
    v_w, v_b.reshape(1, 3), pi_w.reshape(1, 32), pi_b.reshape(1, 1))


def kernel(x, edge_index, batch_size, gin_W1, gin_b1, gin_bn_g, gin_bn_b,
           gin_W2, gin_b2, norm_g, norm_b, jk_W, jk_b, lin_W, lin_b,
           v_W, v_b, pi_W, pi_b):
  pad = EPC - E
  ar = jnp.arange(pad, dtype=jnp.int32)
  src2d = jnp.concatenate([edge_index[0], ar % N])
  dst2d = jnp.concatenate([edge_index[1], N + (ar % (NP - N))])
  h = x
  xs = []
  for i in range(DEPTH):
    parts = _seg_partials(h, src2d, dst2d)
    h = _layer_tc(h, parts, gin_W1[i], gin_b1[i],
                  gin_bn_g[i], gin_bn_b[i], gin_W2[i], gin_b2[i],
                  norm_g[i], norm_b[i])
    xs.append(h)
  pi, v = _head_tc(x, xs[0], xs[1], xs[2], jk_W, jk_b, lin_W, lin_b,
                   v_W, v_b, pi_W, pi_b)
  return (pi.reshape(BATCH, GPN), v.reshape(BATCH, 3))
